# Initial kernel scaffold; baseline (speedup 1.0000x reference)
import jax
import jax.numpy as jnp
from jax.experimental import pallas as pl


def kernel(x, gamma):
    raise NotImplementedError("write your pallas kernel here")


# speedup vs baseline: 1.3806x; 1.3806x over previous
import jax
import jax.numpy as jnp
from jax.experimental import pallas as pl
from jax.experimental.pallas import tpu as pltpu

_R = 2
_KS = 2 * _R + 1
_K2 = _KS * _KS
_TOPK = 10
_TAU = 0.1
_EPS = 1e-8
_TH = 8


def _ppm_body(gamma_ref, xp_hbm, o_ref, slab, sc, sem):
    th = _TH
    C = o_ref.shape[1]
    W = o_ref.shape[3]
    f32 = jnp.float32
    rows = th + 2 * _R
    b = pl.program_id(0)
    t = pl.program_id(1)

    cp = pltpu.make_async_copy(
        xp_hbm.at[b, :, pl.ds(t * th, 2 * th), :], slab, sem)
    cp.start()
    cp.wait()

    sn2 = jnp.zeros((rows, W + 2 * _R), f32)
    for c in range(C):
        v = slab[c, 0:rows, :]
        sn2 = sn2 + v * v
    inv = jax.lax.rsqrt(jnp.maximum(sn2, f32(_EPS * _EPS)))
    inv_c = inv[_R:_R + th, _R:_R + W]

    for i in range(_KS):
        accs = [jnp.zeros((th, W), f32) for _ in range(_KS)]
        for c in range(C):
            row8 = slab[c, i:i + th, :]
            ctr = slab[c, _R:_R + th, _R:_R + W]
            for j in range(_KS):
                accs[j] = accs[j] + row8[:, j:j + W] * ctr
        for j in range(_KS):
            sc[i * _KS + j] = accs[j] * (inv[i:i + th, j:j + W] * inv_c) \
                * f32(1.0 / _TAU)

    mx = sc[0]
    for o in range(1, _K2):
        mx = jnp.maximum(mx, sc[o])
    neg = jnp.full((th, W), -jnp.inf, f32)
    thr = mx
    for _ in range(_TOPK - 1):
        nm = neg
        for o in range(_K2):
            s = sc[o]
            nm = jnp.maximum(nm, jnp.where(s < thr, s, neg))
        thr = nm

    den = jnp.zeros((th, W), f32)
    for o in range(_K2):
        s = sc[o]
        e = jnp.where(s >= thr, jnp.exp(s - mx), f32(0.0))
        sc[o] = e
        den = den + e
    scale = gamma_ref[0] / den

    for c in range(C):
        acc = jnp.zeros((th, W), f32)
        o = 0
        for i in range(_KS):
            for j in range(_KS):
                acc = acc + sc[o] * slab[c, i:i + th, j:j + W]
                o += 1
        o_ref[0, c] = slab[c, _R:_R + th, _R:_R + W] + scale * acc


def kernel(x, gamma):
    B, C, H, W = x.shape
    xp = jnp.pad(x, ((0, 0), (0, 0), (_R, _TH - _R), (_R, _R)))
    g = jnp.reshape(gamma, (1,)).astype(x.dtype)
    out = pl.pallas_call(
        _ppm_body,
        grid=(B, H // _TH),
        in_specs=[
            pl.BlockSpec(memory_space=pltpu.SMEM),
            pl.BlockSpec(memory_space=pl.ANY),
        ],
        out_specs=pl.BlockSpec((1, C, _TH, W), lambda b, t: (b, 0, t, 0)),
        out_shape=jax.ShapeDtypeStruct((B, C, H, W), x.dtype),
        scratch_shapes=[
            pltpu.VMEM((C, 2 * _TH, W + 2 * _R), jnp.float32),
            pltpu.VMEM((_K2, _TH, W), jnp.float32),
            pltpu.SemaphoreType.DMA,
        ],
    )(g, xp)
    return out


# speedup vs baseline: 4.3577x; 3.1563x over previous
import jax
import jax.numpy as jnp
from jax.experimental import pallas as pl
from jax.experimental.pallas import tpu as pltpu

_R = 2
_KS = 2 * _R + 1
_K2 = _KS * _KS
_CTR = _K2 // 2
_TOPK = 10
_TAU = 0.1
_EPS = 1e-8
_TH = 16
_SLAB = _TH + 8


def _ppm_body(gamma_ref, xp_hbm, o_ref, slab, sh, sc, sem):
    th = _TH
    C = o_ref.shape[1]
    W = o_ref.shape[3]
    f32 = jnp.float32
    b = pl.program_id(0)
    t = pl.program_id(1)
    nt = pl.num_programs(1)
    step = b * nt + t
    cur = jax.lax.rem(step, 2)
    nxt = 1 - cur

    def band_copy(bb, tt, buf, s):
        return pltpu.make_async_copy(
            xp_hbm.at[bb, :, pl.ds(tt * th, _SLAB), :], slab.at[buf],
            sem.at[s])

    @pl.when(step == 0)
    def _():
        band_copy(0, 0, 0, 0).start()

    band_copy(b, t, cur, cur).wait()

    @pl.when(step < pl.num_programs(0) * nt - 1)
    def _():
        nb = jnp.where(t == nt - 1, b + 1, b)
        ntt = jnp.where(t == nt - 1, 0, t + 1)
        band_copy(nb, ntt, nxt, nxt).start()

    cs = slab.at[cur]

    sn2f = jnp.zeros((_SLAB, W + 2 * _R), f32)
    for c in range(C):
        v = cs[c]
        sn2f = sn2f + v * v
        for j in range(_KS):
            vj = v[:, j:j + W]
            for i in range(_KS):
                sh[i * _KS + j, c] = vj[i:i + th, :]
    inv = jax.lax.rsqrt(jnp.maximum(sn2f, f32(_EPS * _EPS)))
    inv_c = inv[_R:_R + th, _R:_R + W]

    for i in range(_KS):
        js = [j for j in range(_KS) if i * _KS + j != _CTR]
        accs = {j: jnp.zeros((th, W), f32) for j in js}
        for c in range(C):
            ctr = sh[_CTR, c]
            for j in js:
                accs[j] = accs[j] + sh[i * _KS + j, c] * ctr
        for j in js:
            sc[i * _KS + j] = accs[j] * (inv[i:i + th, j:j + W] * inv_c) \
                * f32(1.0 / _TAU)
    sc[_CTR] = jnp.full((th, W), 1.0 / _TAU, f32)

    mx = sc[0]
    for o in range(1, _K2):
        mx = jnp.maximum(mx, sc[o])
    neg = jnp.full((th, W), -jnp.inf, f32)
    thr = mx
    for _ in range(_TOPK - 1):
        nm = neg
        for o in range(_K2):
            s = sc[o]
            nm = jnp.maximum(nm, jnp.where(s < thr, s, neg))
        thr = nm

    den = jnp.zeros((th, W), f32)
    for o in range(_K2):
        s = sc[o]
        e = jnp.where(s >= thr, jnp.exp(s - mx), f32(0.0))
        sc[o] = e
        den = den + e
    scale = gamma_ref[0] / den

    for i in range(_KS):
        ws = [sc[i * _KS + j] for j in range(_KS)]
        for c in range(C):
            acc = o_ref[0, c] if i > 0 else jnp.zeros((th, W), f32)
            for j in range(_KS):
                acc = acc + ws[j] * sh[i * _KS + j, c]
            o_ref[0, c] = acc
    for c in range(C):
        o_ref[0, c] = sh[_CTR, c] + scale * o_ref[0, c]


def kernel(x, gamma):
    B, C, H, W = x.shape
    xp = jnp.pad(x, ((0, 0), (0, 0), (_R, _SLAB - _TH - _R), (_R, _R)))
    g = jnp.reshape(gamma, (1,)).astype(x.dtype)
    out = pl.pallas_call(
        _ppm_body,
        grid=(B, H // _TH),
        in_specs=[
            pl.BlockSpec(memory_space=pltpu.SMEM),
            pl.BlockSpec(memory_space=pl.ANY),
        ],
        out_specs=pl.BlockSpec((1, C, _TH, W), lambda b, t: (b, 0, t, 0)),
        out_shape=jax.ShapeDtypeStruct((B, C, H, W), x.dtype),
        scratch_shapes=[
            pltpu.VMEM((2, C, _SLAB, W + 2 * _R), jnp.float32),
            pltpu.VMEM((_K2, C, _TH, W), jnp.float32),
            pltpu.VMEM((_K2, _TH, W), jnp.float32),
            pltpu.SemaphoreType.DMA((2,)),
        ],
    )(g, xp)
    return out
